# Initial kernel scaffold; baseline (speedup 1.0000x reference)
import jax
import jax.numpy as jnp
from jax import lax
from jax.experimental import pallas as pl
from jax.experimental.pallas import tpu as pltpu
from jax.experimental.pallas import tpu_sc as plsc

N_ROWS = 100000
D = 512
S = 256
NC = 2
NS = 16
LANES = 16
ROWS_PER_SUB = N_ROWS // NS
CHUNK = 125
NCHUNK = ROWS_PER_SUB // CHUNK
DH = D // NC
FVEC = DH // LANES
SEG_PER_SUB = S // NS


def _seg_mean_body(x_hbm, ids_hbm, out_hbm,
                   xbuf, ids_v, ones_v, tmp_v, cnt_v, sums_sh, cnt_sh):
    c = lax.axis_index("c")
    s = lax.axis_index("s")

    zero = jnp.zeros((LANES,), jnp.float32)
    one = jnp.ones((LANES,), jnp.float32)

    @pl.loop(0, SEG_PER_SUB)
    def _(i):
        @pl.loop(0, FVEC)
        def _(f):
            tmp_v[i, pl.ds(f * LANES, LANES)] = zero

    @pl.loop(0, CHUNK)
    def _(i):
        ones_v[i, :] = one

    @pl.loop(0, SEG_PER_SUB)
    def _(i):
        cnt_v[i, :] = zero

    pltpu.sync_copy(tmp_v, sums_sh.at[pl.ds(s * SEG_PER_SUB, SEG_PER_SUB)])
    pltpu.sync_copy(cnt_v, cnt_sh.at[pl.ds(s * SEG_PER_SUB, SEG_PER_SUB)])

    pltpu.sync_copy(ids_hbm.at[s], ids_v)

    plsc.subcore_barrier()

    @pl.loop(0, NCHUNK)
    def _(j):
        pltpu.sync_copy(x_hbm.at[s, j, :, pl.ds(c * DH, DH)], xbuf)
        pltpu.sync_copy(xbuf, sums_sh.at[ids_v.at[j]], add=True)
        pltpu.sync_copy(ones_v, cnt_sh.at[ids_v.at[j]], add=True)

    plsc.subcore_barrier()

    pltpu.sync_copy(sums_sh.at[pl.ds(s * SEG_PER_SUB, SEG_PER_SUB)],
                    xbuf.at[pl.ds(0, SEG_PER_SUB)])
    pltpu.sync_copy(cnt_sh.at[pl.ds(s * SEG_PER_SUB, SEG_PER_SUB)], cnt_v)

    @pl.loop(0, SEG_PER_SUB)
    def _(r):
        denom = jnp.maximum(cnt_v[r, :], 1.0)

        @pl.loop(0, FVEC)
        def _(f):
            tmp_v[r, pl.ds(f * LANES, LANES)] = (
                xbuf[r, pl.ds(f * LANES, LANES)] / denom)

    pltpu.sync_copy(
        tmp_v,
        out_hbm.at[pl.ds(s * SEG_PER_SUB, SEG_PER_SUB), pl.ds(c * DH, DH)])


def kernel(x, segment_ids):
    ids = segment_ids.astype(jnp.int32).reshape(NS, NCHUNK, CHUNK)
    x4 = x.reshape(NS, NCHUNK, CHUNK, D)
    mesh = plsc.VectorSubcoreMesh(core_axis_name="c", subcore_axis_name="s")
    kfn = pl.kernel(
        _seg_mean_body,
        out_type=jax.ShapeDtypeStruct((S, D), jnp.float32),
        mesh=mesh,
        scratch_types=[
            pltpu.VMEM((CHUNK, DH), jnp.float32),
            pltpu.VMEM((NCHUNK, CHUNK), jnp.int32),
            pltpu.VMEM((CHUNK, LANES), jnp.float32),
            pltpu.VMEM((SEG_PER_SUB, DH), jnp.float32),
            pltpu.VMEM((SEG_PER_SUB, LANES), jnp.float32),
            pltpu.VMEM_SHARED((S, DH), jnp.float32),
            pltpu.VMEM_SHARED((S, LANES), jnp.float32),
        ],
    )
    return kfn(x4, ids)


# speedup vs baseline: 3.6537x; 3.6537x over previous
import dataclasses

import jax
import jax.numpy as jnp
from jax import lax
from jax.experimental import pallas as pl
from jax.experimental.pallas import tpu as pltpu
from jax.experimental.pallas import tpu_sc as plsc

N_ROWS = 100000
D = 512
S = 256
NC = 2
NS = 16
LANES = 16
CH = 128
DH = D // NC
FVEC = DH // LANES
SEG_PER_SUB = S // NS
OFFS_PAD = 272


def _seg_mean_body(x_hbm, offs_hbm, out_hbm, xb, offs_v, acc_v):
    c = lax.axis_index("c")
    s = lax.axis_index("s")

    zero = jnp.zeros((LANES,), jnp.float32)
    lane = lax.iota(jnp.int32, 16)

    pltpu.sync_copy(offs_hbm, offs_v)

    w0 = offs_v[pl.ds(s * SEG_PER_SUB, LANES)]
    w1 = offs_v[pl.ds(s * SEG_PER_SUB + 1, LANES)]

    def extract(vec, t):
        return jnp.sum(jnp.where(lane == t, vec, 0))

    seg_lo = [extract(w0, t) for t in range(SEG_PER_SUB)]
    seg_hi = [extract(w1, t) for t in range(SEG_PER_SUB)]
    row_start = seg_lo[0]
    row_end = seg_hi[SEG_PER_SUB - 1]

    @pl.loop(0, SEG_PER_SUB)
    def _(i):
        @pl.loop(0, FVEC)
        def _(f):
            acc_v[i, pl.ds(f * LANES, LANES)] = zero

    a8 = (row_start // 8) * 8
    nk = (row_end - a8 + CH - 1) // CH

    def chunk_body(k, carry):
        base = jnp.minimum(a8 + k * CH, N_ROWS - CH)
        pltpu.sync_copy(x_hbm.at[pl.ds(base, CH), pl.ds(c * DH, DH)], xb)
        win_lo = jnp.maximum(row_start, a8 + k * CH)
        win_hi = jnp.minimum(row_end, a8 + (k + 1) * CH)
        for t in range(SEG_PER_SUB):
            a = jnp.maximum(seg_lo[t], win_lo) - base
            b = jnp.minimum(seg_hi[t], win_hi) - base

            def row_body(r, regs):
                return tuple(regs[f] + xb[r, pl.ds(f * LANES, LANES)]
                             for f in range(FVEC))

            regs0 = tuple(zero for _ in range(FVEC))
            regs = lax.fori_loop(a, b, row_body, regs0)
            for f in range(FVEC):
                plsc.addupdate(acc_v.at[t, pl.ds(f * LANES, LANES)], regs[f])
        return carry

    lax.fori_loop(0, nk, chunk_body, 0)

    for t in range(SEG_PER_SUB):
        cnt = seg_hi[t] - seg_lo[t]
        denom = jnp.broadcast_to(
            jnp.maximum(cnt.astype(jnp.float32), 1.0), (LANES,))
        for f in range(FVEC):
            acc_v[t, pl.ds(f * LANES, LANES)] = (
                acc_v[t, pl.ds(f * LANES, LANES)] / denom)

    pltpu.sync_copy(
        acc_v,
        out_hbm.at[pl.ds(s * SEG_PER_SUB, SEG_PER_SUB), pl.ds(c * DH, DH)])


def kernel(x, segment_ids):
    ids32 = segment_ids.astype(jnp.int32)
    offs = jnp.searchsorted(
        ids32, jnp.arange(S + 1, dtype=jnp.int32), side="left")
    offs = jnp.pad(offs.astype(jnp.int32), (0, OFFS_PAD - (S + 1)))
    mesh = plsc.VectorSubcoreMesh(core_axis_name="c", subcore_axis_name="s")
    cp = pltpu.CompilerParams()
    if "needs_layout_passes" in pltpu.CompilerParams.__dataclass_fields__:
        cp = dataclasses.replace(cp, needs_layout_passes=False)
    kfn = pl.kernel(
        _seg_mean_body,
        out_type=jax.ShapeDtypeStruct((S, D), jnp.float32),
        mesh=mesh,
        compiler_params=cp,
        scratch_types=[
            pltpu.VMEM((CH, DH), jnp.float32),
            pltpu.VMEM((OFFS_PAD,), jnp.int32),
            pltpu.VMEM((SEG_PER_SUB, DH), jnp.float32),
        ],
    )
    return kfn(x, offs)


# speedup vs baseline: 5.0336x; 1.3777x over previous
import dataclasses

import jax
import jax.numpy as jnp
from jax import lax
from jax.experimental import pallas as pl
from jax.experimental.pallas import tpu as pltpu
from jax.experimental.pallas import tpu_sc as plsc

N_ROWS = 100000
D = 512
S = 256
NC = 2
NS = 16
LANES = 16
CH = 192
DH = D // NC
FVEC = DH // LANES
SEG_PER_SUB = S // NS
OFFS_PAD = 272


def _seg_mean_body(x_hbm, offs_hbm, out_hbm, xb0, xb1, offs_v, acc_v,
                   sem0, sem1):
    c = lax.axis_index("c")
    s = lax.axis_index("s")

    zero = jnp.zeros((LANES,), jnp.float32)
    lane = lax.iota(jnp.int32, 16)

    pltpu.sync_copy(offs_hbm, offs_v)

    w0 = offs_v[pl.ds(s * SEG_PER_SUB, LANES)]
    w1 = offs_v[pl.ds(s * SEG_PER_SUB + 1, LANES)]

    def extract(vec, t):
        return jnp.sum(jnp.where(lane == t, vec, 0))

    seg_lo = [extract(w0, t) for t in range(SEG_PER_SUB)]
    seg_hi = [extract(w1, t) for t in range(SEG_PER_SUB)]
    row_start = seg_lo[0]
    row_end = seg_hi[SEG_PER_SUB - 1]

    @pl.loop(0, SEG_PER_SUB)
    def _(i):
        @pl.loop(0, FVEC)
        def _(f):
            acc_v[i, pl.ds(f * LANES, LANES)] = zero

    a8 = (row_start // 8) * 8
    nk = (row_end - a8 + CH - 1) // CH

    def chunk_base(k):
        return jnp.minimum(a8 + k * CH, N_ROWS - CH)

    def start(k, xb, sem):
        pltpu.async_copy(
            x_hbm.at[pl.ds(chunk_base(k), CH), pl.ds(c * DH, DH)], xb, sem)

    def wait(xb, sem):
        pltpu.make_async_copy(
            x_hbm.at[pl.ds(0, CH), pl.ds(0, DH)], xb, sem).wait()

    def process(k, xb):
        base = chunk_base(k)
        win_lo = jnp.maximum(row_start, a8 + k * CH)
        win_hi = jnp.minimum(row_end, a8 + (k + 1) * CH)
        for t in range(SEG_PER_SUB):
            a = jnp.maximum(seg_lo[t], win_lo) - base
            b = jnp.minimum(seg_hi[t], win_hi) - base

            def row_body(r, regs):
                return tuple(regs[f] + xb[r, pl.ds(f * LANES, LANES)]
                             for f in range(FVEC))

            regs0 = tuple(zero for _ in range(FVEC))
            regs = lax.fori_loop(a, b, row_body, regs0)
            for f in range(FVEC):
                plsc.addupdate(acc_v.at[t, pl.ds(f * LANES, LANES)], regs[f])

    @pl.when(nk > 0)
    def _():
        start(0, xb0, sem0)

    def pair_body(m, carry):
        k0 = 2 * m

        @pl.when(k0 + 1 < nk)
        def _():
            start(k0 + 1, xb1, sem1)

        wait(xb0, sem0)
        process(k0, xb0)

        @pl.when(k0 + 2 < nk)
        def _():
            start(k0 + 2, xb0, sem0)

        @pl.when(k0 + 1 < nk)
        def _():
            wait(xb1, sem1)
            process(k0 + 1, xb1)

        return carry

    lax.fori_loop(0, (nk + 1) // 2, pair_body, 0)

    for t in range(SEG_PER_SUB):
        cnt = seg_hi[t] - seg_lo[t]
        denom = jnp.broadcast_to(
            jnp.maximum(cnt.astype(jnp.float32), 1.0), (LANES,))
        for f in range(FVEC):
            acc_v[t, pl.ds(f * LANES, LANES)] = (
                acc_v[t, pl.ds(f * LANES, LANES)] / denom)

    pltpu.sync_copy(
        acc_v,
        out_hbm.at[pl.ds(s * SEG_PER_SUB, SEG_PER_SUB), pl.ds(c * DH, DH)])


def kernel(x, segment_ids):
    ids32 = segment_ids.astype(jnp.int32)
    offs = jnp.searchsorted(
        ids32, jnp.arange(S + 1, dtype=jnp.int32), side="left")
    offs = jnp.pad(offs.astype(jnp.int32), (0, OFFS_PAD - (S + 1)))
    mesh = plsc.VectorSubcoreMesh(core_axis_name="c", subcore_axis_name="s")
    cp = pltpu.CompilerParams()
    if "needs_layout_passes" in pltpu.CompilerParams.__dataclass_fields__:
        cp = dataclasses.replace(cp, needs_layout_passes=False)
    kfn = pl.kernel(
        _seg_mean_body,
        out_type=jax.ShapeDtypeStruct((S, D), jnp.float32),
        mesh=mesh,
        compiler_params=cp,
        scratch_types=[
            pltpu.VMEM((CH, DH), jnp.float32),
            pltpu.VMEM((CH, DH), jnp.float32),
            pltpu.VMEM((OFFS_PAD,), jnp.int32),
            pltpu.VMEM((SEG_PER_SUB, DH), jnp.float32),
            pltpu.SemaphoreType.DMA,
            pltpu.SemaphoreType.DMA,
        ],
    )
    return kfn(x, offs)
